# baseline (speedup 1.0000x reference)
import functools

import jax
import jax.numpy as jnp
from jax import lax
from jax.experimental import pallas as pl
from jax.experimental.pallas import tpu as pltpu
from jax.experimental.pallas import tpu_sc as plsc

N = 10000
E = 160000
D_IN = 256
HID = 128
LAT = 64

NC = 2
NS = 16
L = 16
NW = NC * NS

N_PAD = 10240
E_PAD = 163840
CHUNK = 128
EPW = E_PAD // NW
NCHUNK = EPW // CHUNK
ROWS_PW = N_PAD // NS
FLUSH = 128

_MESH = plsc.VectorSubcoreMesh(
    core_axis_name="c", subcore_axis_name="s", num_cores=NC, num_subcores=NS)


EROWS = E // CHUNK
RROWS = EROWS - (NW - 1) * NCHUNK


def _stage_edges(ei_hbm, wid, idx_v, which, want_src):
    @pl.when(wid < NW - 1)
    def _():
        pltpu.sync_copy(ei_hbm.at[which, pl.ds(wid * NCHUNK, NCHUNK)], idx_v)

    @pl.when(wid == NW - 1)
    def _():
        pltpu.sync_copy(
            ei_hbm.at[which, pl.ds((NW - 1) * NCHUNK, RROWS)],
            idx_v.at[pl.ds(0, RROWS)])
        iota = lax.iota(jnp.int32, L)
        kl = CHUNK // L

        def pbody(i, carry):
            vals = i * L + iota
            if not want_src:
                vals = N + (vals & 127)
            idx_v[i // kl, pl.ds((i % kl) * L, L)] = vals
            return carry
        lax.fori_loop(RROWS * kl, NCHUNK * kl, pbody, 0)


def _stage_flat(eif_hbm, wid, v, which, want_src):
    iota = lax.iota(jnp.int32, L)
    epw_real = E - (NW - 1) * EPW

    @pl.when(wid < NW - 1)
    def _():
        pltpu.sync_copy(eif_hbm.at[which, pl.ds(wid * EPW, EPW)], v)

    @pl.when(wid == NW - 1)
    def _():
        pltpu.sync_copy(eif_hbm.at[which, pl.ds((NW - 1) * EPW, epw_real)],
                        v.at[pl.ds(0, epw_real)])

        def pbody(i, carry):
            vals = i * L + iota
            if not want_src:
                vals = N + (vals & 127)
            v[pl.ds(i * L, L)] = vals
            return carry
        lax.fori_loop(epw_real // L, EPW // L, pbody, 0)


def _deg_body(ei_hbm, out0, out1, dst_v, hist_v, acc_v, tmp_v, shared):
    cid = lax.axis_index("c")
    sid = lax.axis_index("s")
    wid = cid * NS + sid
    _stage_flat(ei_hbm, wid, dst_v, 1, False)

    zero = jnp.zeros((L,), jnp.float32)
    ones = jnp.ones((L,), jnp.float32)

    def zbody(i, carry):
        hist_v[pl.ds(i * L, L)] = zero
        return carry
    lax.fori_loop(0, N_PAD // L, zbody, 0)

    def hbody(i, carry):
        idx = dst_v[pl.ds(i * L, L)]
        plsc.addupdate_scatter(hist_v, [idx], ones)
        return carry
    lax.fori_loop(0, EPW // L, hbody, 0)

    pltpu.sync_copy(hist_v, shared.at[sid])
    plsc.subcore_barrier()

    base = sid * ROWS_PW
    pltpu.sync_copy(shared.at[0, pl.ds(base, ROWS_PW)], acc_v)
    for t in range(1, NS):
        pltpu.sync_copy(shared.at[t, pl.ds(base, ROWS_PW)], tmp_v)

        def abody(i, carry):
            acc_v[pl.ds(i * L, L)] = acc_v[pl.ds(i * L, L)] + tmp_v[pl.ds(i * L, L)]
            return carry
        lax.fori_loop(0, ROWS_PW // L, abody, 0)

    @pl.when(cid == 0)
    def _():
        pltpu.sync_copy(acc_v, out0.at[pl.ds(base, ROWS_PW)])

    @pl.when(cid == 1)
    def _():
        pltpu.sync_copy(acc_v, out1.at[pl.ds(base, ROWS_PW)])


_deg_kernel = functools.partial(
    pl.kernel, mesh=_MESH,
    compiler_params=pltpu.CompilerParams(needs_layout_passes=False),
    out_type=(jax.ShapeDtypeStruct((N_PAD,), jnp.float32),
              jax.ShapeDtypeStruct((N_PAD,), jnp.float32)),
    scratch_types=[
        pltpu.VMEM((EPW,), jnp.int32),
        pltpu.VMEM((N_PAD,), jnp.float32),
        pltpu.VMEM((ROWS_PW,), jnp.float32),
        pltpu.VMEM((ROWS_PW,), jnp.float32),
        pltpu.VMEM_SHARED((NS, N_PAD), jnp.float32),
    ])(_deg_body)


def _make_scatter(D, nbuf):
    DL = D // L

    def body(g_hbm, eif_hbm, out0, out1, src_v, dst_v, *rest):
        gbufs = rest[:nbuf]
        sems = rest[nbuf:2 * nbuf]
        acc = rest[2 * nbuf]
        cid = lax.axis_index("c")
        sid = lax.axis_index("s")
        wid = cid * NS + sid
        _stage_flat(eif_hbm, wid, src_v, 0, True)
        _stage_flat(eif_hbm, wid, dst_v, 1, False)

        zero = jnp.zeros((L,), jnp.float32)
        gb0 = gbufs[0].at[pl.ds(0, FLUSH)]

        def zbody(i, carry):
            gb0[i // DL, pl.ds((i % DL) * L, L)] = zero
            return carry
        lax.fori_loop(0, FLUSH * DL, zbody, 0)

        base = sid * ROWS_PW
        for h in range(ROWS_PW // FLUSH):
            pltpu.sync_copy(gb0, acc.at[pl.ds(base + h * FLUSH, FLUSH)])
        plsc.subcore_barrier()

        GC = 2 * CHUNK
        NT = NCHUNK // 2
        for b in range(nbuf):
            pltpu.async_copy(g_hbm.at[src_v.at[pl.ds(GC * b, GC)]],
                             gbufs[b], sems[b])

        def step(j, carry):
            for b in range(nbuf):
                t = nbuf * j + b
                pltpu.make_async_copy(
                    g_hbm.at[src_v.at[pl.ds(GC * t, GC)]], gbufs[b], sems[b]).wait()
                pltpu.sync_copy(gbufs[b],
                                acc.at[dst_v.at[pl.ds(GC * t, GC)]], add=True)

                @pl.when(t + nbuf < NT)
                def _():
                    pltpu.async_copy(
                        g_hbm.at[src_v.at[pl.ds(GC * (t + nbuf), GC)]],
                        gbufs[b], sems[b])
            return carry
        lax.fori_loop(0, NT // nbuf, step, 0)

        plsc.subcore_barrier()
        for h in range(ROWS_PW // FLUSH):
            pltpu.sync_copy(acc.at[pl.ds(base + h * FLUSH, FLUSH)], gb0)

            @pl.when(cid == 0)
            def _():
                pltpu.sync_copy(gb0, out0.at[pl.ds(base + h * FLUSH, FLUSH)])

            @pl.when(cid == 1)
            def _():
                pltpu.sync_copy(gb0, out1.at[pl.ds(base + h * FLUSH, FLUSH)])

    return functools.partial(
        pl.kernel, mesh=_MESH,
        compiler_params=pltpu.CompilerParams(use_tc_tiling_on_sc=False),
        out_type=(jax.ShapeDtypeStruct((N_PAD, D), jnp.float32),
                  jax.ShapeDtypeStruct((N_PAD, D), jnp.float32)),
        scratch_types=[
            pltpu.VMEM((EPW,), jnp.int32),
            pltpu.VMEM((EPW,), jnp.int32),
        ] + [pltpu.VMEM((2 * CHUNK, D), jnp.float32)] * nbuf
          + [pltpu.SemaphoreType.DMA] * nbuf
          + [pltpu.VMEM_SHARED((N_PAD, D), jnp.float32)],
    )(body)


_scatter_hid = _make_scatter(HID, nbuf=1)
_scatter_lat = _make_scatter(LAT, nbuf=2)


RB = 1024
_GRID = (N_PAD // RB,)


def _rows(d):
    return pl.BlockSpec((RB, d), lambda i: (i, 0))


def _vec():
    return pl.BlockSpec((RB,), lambda i: (i,))


def _full(*shape):
    return pl.BlockSpec(shape, lambda i: tuple(0 for _ in shape))


def _mm1_body(x_ref, w_ref, d0_ref, d1_ref, g_ref, dinv_ref):
    dinv = lax.rsqrt(d0_ref[:] + d1_ref[:] + 1.0)
    h = jnp.dot(x_ref[:], w_ref[:], preferred_element_type=jnp.float32)
    g_ref[:] = h * dinv[:, None]
    dinv_ref[:] = dinv


_mm1 = pl.pallas_call(
    _mm1_body,
    grid=_GRID,
    in_specs=[_rows(D_IN), _full(D_IN, HID), _vec(), _vec()],
    out_specs=(_rows(HID), _vec()),
    out_shape=(jax.ShapeDtypeStruct((N_PAD, HID), jnp.float32),
               jax.ShapeDtypeStruct((N_PAD,), jnp.float32)),
)


def _mid_body(s0_ref, s1_ref, g1_ref, dinv_ref, b1_ref, w2_ref, g2_ref):
    dinv = dinv_ref[:]
    out1 = (s0_ref[:] + s1_ref[:] + g1_ref[:]) * dinv[:, None] + b1_ref[:]
    x2 = jnp.maximum(out1, 0.0)
    h2 = jnp.dot(x2, w2_ref[:], preferred_element_type=jnp.float32)
    g2_ref[:] = h2 * dinv[:, None]


_mid = pl.pallas_call(
    _mid_body,
    grid=_GRID,
    in_specs=[_rows(HID), _rows(HID), _rows(HID), _vec(),
              _full(1, HID), _full(HID, LAT)],
    out_specs=_rows(LAT),
    out_shape=jax.ShapeDtypeStruct((N_PAD, LAT), jnp.float32),
)


def _fin_body(s0_ref, s1_ref, g2_ref, dinv_ref, b2_ref, mean_ref, logvar_ref):
    dinv = dinv_ref[:]
    out = (s0_ref[:] + s1_ref[:] + g2_ref[:]) * dinv[:, None] + b2_ref[:]
    mean_ref[:] = out[:, : LAT // 2]
    logvar_ref[:] = out[:, LAT // 2:]


_fin = pl.pallas_call(
    _fin_body,
    grid=_GRID,
    in_specs=[_rows(LAT), _rows(LAT), _rows(LAT), _vec(), _full(1, LAT)],
    out_specs=(pl.BlockSpec((RB, LAT // 2), lambda i: (i, 0)),
               pl.BlockSpec((RB, LAT // 2), lambda i: (i, 0))),
    out_shape=(jax.ShapeDtypeStruct((N, LAT // 2), jnp.float32),
               jax.ShapeDtypeStruct((N, LAT // 2), jnp.float32)),
)


def kernel(x, edge_index, W1, b1, W2, b2):
    ei = edge_index.astype(jnp.int32)

    deg0, deg1 = _deg_kernel(ei)
    g1, dinv = _mm1(x, W1, deg0, deg1)
    s1a, s1b = _scatter_hid(g1, ei)
    g2 = _mid(s1a, s1b, g1, dinv, b1.reshape(1, HID), W2)
    s2a, s2b = _scatter_lat(g2, ei)
    return _fin(s2a, s2b, g2, dinv, b2.reshape(1, LAT))


# speedup vs baseline: 1.1218x; 1.1218x over previous
import functools

import jax
import jax.numpy as jnp
from jax import lax
from jax.experimental import pallas as pl
from jax.experimental.pallas import tpu as pltpu
from jax.experimental.pallas import tpu_sc as plsc

N = 10000
E = 160000
D_IN = 256
HID = 128
LAT = 64

NC = 2
NS = 16
L = 16
NW = NC * NS

N_PAD = 10240
E_PAD = 163840
CHUNK = 128
EPW = E_PAD // NW
NCHUNK = EPW // CHUNK
ROWS_PW = N_PAD // NS
FLUSH = 128

_MESH = plsc.VectorSubcoreMesh(
    core_axis_name="c", subcore_axis_name="s", num_cores=NC, num_subcores=NS)


EROWS = E // CHUNK
RROWS = EROWS - (NW - 1) * NCHUNK


def _stage_edges(ei_hbm, wid, idx_v, which, want_src):
    @pl.when(wid < NW - 1)
    def _():
        pltpu.sync_copy(ei_hbm.at[which, pl.ds(wid * NCHUNK, NCHUNK)], idx_v)

    @pl.when(wid == NW - 1)
    def _():
        pltpu.sync_copy(
            ei_hbm.at[which, pl.ds((NW - 1) * NCHUNK, RROWS)],
            idx_v.at[pl.ds(0, RROWS)])
        iota = lax.iota(jnp.int32, L)
        kl = CHUNK // L

        def pbody(i, carry):
            vals = i * L + iota
            if not want_src:
                vals = N + (vals & 127)
            idx_v[i // kl, pl.ds((i % kl) * L, L)] = vals
            return carry
        lax.fori_loop(RROWS * kl, NCHUNK * kl, pbody, 0)


def _stage_flat(eif_hbm, wid, v, which, want_src):
    iota = lax.iota(jnp.int32, L)
    epw_real = E - (NW - 1) * EPW

    @pl.when(wid < NW - 1)
    def _():
        pltpu.sync_copy(eif_hbm.at[which, pl.ds(wid * EPW, EPW)], v)

    @pl.when(wid == NW - 1)
    def _():
        pltpu.sync_copy(eif_hbm.at[which, pl.ds((NW - 1) * EPW, epw_real)],
                        v.at[pl.ds(0, epw_real)])

        def pbody(i, carry):
            vals = i * L + iota
            if not want_src:
                vals = N + (vals & 127)
            v[pl.ds(i * L, L)] = vals
            return carry
        lax.fori_loop(epw_real // L, EPW // L, pbody, 0)


def _deg_body(ei_hbm, out0, out1, dst_v, hist_v, acc_v, tmp_v, shared):
    cid = lax.axis_index("c")
    sid = lax.axis_index("s")
    wid = cid * NS + sid
    _stage_flat(ei_hbm, wid, dst_v, 1, False)

    zero = jnp.zeros((L,), jnp.float32)
    ones = jnp.ones((L,), jnp.float32)

    def zbody(i, carry):
        hist_v[pl.ds(i * L, L)] = zero
        return carry
    lax.fori_loop(0, N_PAD // L, zbody, 0)

    def hbody(i, carry):
        idx = dst_v[pl.ds(i * L, L)]
        plsc.addupdate_scatter(hist_v, [idx], ones)
        return carry
    lax.fori_loop(0, EPW // L, hbody, 0)

    pltpu.sync_copy(hist_v, shared.at[sid])
    plsc.subcore_barrier()

    base = sid * ROWS_PW
    pltpu.sync_copy(shared.at[0, pl.ds(base, ROWS_PW)], acc_v)
    for t in range(1, NS):
        pltpu.sync_copy(shared.at[t, pl.ds(base, ROWS_PW)], tmp_v)

        def abody(i, carry):
            acc_v[pl.ds(i * L, L)] = acc_v[pl.ds(i * L, L)] + tmp_v[pl.ds(i * L, L)]
            return carry
        lax.fori_loop(0, ROWS_PW // L, abody, 0)

    @pl.when(cid == 0)
    def _():
        pltpu.sync_copy(acc_v, out0.at[pl.ds(base, ROWS_PW)])

    @pl.when(cid == 1)
    def _():
        pltpu.sync_copy(acc_v, out1.at[pl.ds(base, ROWS_PW)])


_deg_kernel = functools.partial(
    pl.kernel, mesh=_MESH,
    compiler_params=pltpu.CompilerParams(needs_layout_passes=False),
    out_type=(jax.ShapeDtypeStruct((N_PAD,), jnp.float32),
              jax.ShapeDtypeStruct((N_PAD,), jnp.float32)),
    scratch_types=[
        pltpu.VMEM((EPW,), jnp.int32),
        pltpu.VMEM((N_PAD,), jnp.float32),
        pltpu.VMEM((ROWS_PW,), jnp.float32),
        pltpu.VMEM((ROWS_PW,), jnp.float32),
        pltpu.VMEM_SHARED((NS, N_PAD), jnp.float32),
    ])(_deg_body)


def _make_scatter(D, nbuf):
    DL = D // L

    def body(g_hbm, eif_hbm, out0, out1, src_v, dst_v, *rest):
        gbufs = rest[:nbuf]
        sems = rest[nbuf:2 * nbuf]
        acc = rest[2 * nbuf]
        cid = lax.axis_index("c")
        sid = lax.axis_index("s")
        wid = cid * NS + sid
        _stage_flat(eif_hbm, wid, src_v, 0, True)
        _stage_flat(eif_hbm, wid, dst_v, 1, False)

        zero = jnp.zeros((L,), jnp.float32)
        gb0 = gbufs[0]

        def zbody(i, carry):
            gb0[i // DL, pl.ds((i % DL) * L, L)] = zero
            return carry
        lax.fori_loop(0, FLUSH * DL, zbody, 0)

        base = sid * ROWS_PW
        for h in range(ROWS_PW // FLUSH):
            pltpu.sync_copy(gb0, acc.at[pl.ds(base + h * FLUSH, FLUSH)])
        plsc.subcore_barrier()

        for b in range(nbuf):
            pltpu.async_copy(g_hbm.at[src_v.at[pl.ds(CHUNK * b, CHUNK)]],
                             gbufs[b], sems[b])

        def step(j, carry):
            for b in range(nbuf):
                t = nbuf * j + b
                pltpu.make_async_copy(
                    g_hbm.at[src_v.at[pl.ds(CHUNK * t, CHUNK)]],
                    gbufs[b], sems[b]).wait()
                pltpu.sync_copy(gbufs[b],
                                acc.at[dst_v.at[pl.ds(CHUNK * t, CHUNK)]],
                                add=True)

                @pl.when(t + nbuf < NCHUNK)
                def _():
                    pltpu.async_copy(
                        g_hbm.at[src_v.at[pl.ds(CHUNK * (t + nbuf), CHUNK)]],
                        gbufs[b], sems[b])
            return carry
        lax.fori_loop(0, NCHUNK // nbuf, step, 0)

        plsc.subcore_barrier()
        for h in range(ROWS_PW // FLUSH):
            pltpu.sync_copy(acc.at[pl.ds(base + h * FLUSH, FLUSH)], gb0)

            @pl.when(cid == 0)
            def _():
                pltpu.sync_copy(gb0, out0.at[pl.ds(base + h * FLUSH, FLUSH)])

            @pl.when(cid == 1)
            def _():
                pltpu.sync_copy(gb0, out1.at[pl.ds(base + h * FLUSH, FLUSH)])

    return functools.partial(
        pl.kernel, mesh=_MESH,
        compiler_params=pltpu.CompilerParams(use_tc_tiling_on_sc=False),
        out_type=(jax.ShapeDtypeStruct((N_PAD, D), jnp.float32),
                  jax.ShapeDtypeStruct((N_PAD, D), jnp.float32)),
        scratch_types=[
            pltpu.VMEM((EPW,), jnp.int32),
            pltpu.VMEM((EPW,), jnp.int32),
        ] + [pltpu.VMEM((CHUNK, D), jnp.float32)] * nbuf
          + [pltpu.SemaphoreType.DMA] * nbuf
          + [pltpu.VMEM_SHARED((N_PAD, D), jnp.float32)],
    )(body)


_scatter_hid = _make_scatter(HID, nbuf=2)
_scatter_lat = _make_scatter(LAT, nbuf=4)


RB = 1024
_GRID = (N_PAD // RB,)


def _rows(d):
    return pl.BlockSpec((RB, d), lambda i: (i, 0))


def _vec():
    return pl.BlockSpec((RB,), lambda i: (i,))


def _full(*shape):
    return pl.BlockSpec(shape, lambda i: tuple(0 for _ in shape))


def _mm1_body(x_ref, w_ref, d0_ref, d1_ref, g_ref, dinv_ref):
    dinv = lax.rsqrt(d0_ref[:] + d1_ref[:] + 1.0)
    h = jnp.dot(x_ref[:], w_ref[:], preferred_element_type=jnp.float32)
    g_ref[:] = h * dinv[:, None]
    dinv_ref[:] = dinv


_mm1 = pl.pallas_call(
    _mm1_body,
    grid=_GRID,
    in_specs=[_rows(D_IN), _full(D_IN, HID), _vec(), _vec()],
    out_specs=(_rows(HID), _vec()),
    out_shape=(jax.ShapeDtypeStruct((N_PAD, HID), jnp.float32),
               jax.ShapeDtypeStruct((N_PAD,), jnp.float32)),
)


def _mid_body(s0_ref, s1_ref, g1_ref, dinv_ref, b1_ref, w2_ref, g2_ref):
    dinv = dinv_ref[:]
    out1 = (s0_ref[:] + s1_ref[:] + g1_ref[:]) * dinv[:, None] + b1_ref[:]
    x2 = jnp.maximum(out1, 0.0)
    h2 = jnp.dot(x2, w2_ref[:], preferred_element_type=jnp.float32)
    g2_ref[:] = h2 * dinv[:, None]


_mid = pl.pallas_call(
    _mid_body,
    grid=_GRID,
    in_specs=[_rows(HID), _rows(HID), _rows(HID), _vec(),
              _full(1, HID), _full(HID, LAT)],
    out_specs=_rows(LAT),
    out_shape=jax.ShapeDtypeStruct((N_PAD, LAT), jnp.float32),
)


def _fin_body(s0_ref, s1_ref, g2_ref, dinv_ref, b2_ref, mean_ref, logvar_ref):
    dinv = dinv_ref[:]
    out = (s0_ref[:] + s1_ref[:] + g2_ref[:]) * dinv[:, None] + b2_ref[:]
    mean_ref[:] = out[:, : LAT // 2]
    logvar_ref[:] = out[:, LAT // 2:]


_fin = pl.pallas_call(
    _fin_body,
    grid=_GRID,
    in_specs=[_rows(LAT), _rows(LAT), _rows(LAT), _vec(), _full(1, LAT)],
    out_specs=(pl.BlockSpec((RB, LAT // 2), lambda i: (i, 0)),
               pl.BlockSpec((RB, LAT // 2), lambda i: (i, 0))),
    out_shape=(jax.ShapeDtypeStruct((N, LAT // 2), jnp.float32),
               jax.ShapeDtypeStruct((N, LAT // 2), jnp.float32)),
)


def kernel(x, edge_index, W1, b1, W2, b2):
    ei = edge_index.astype(jnp.int32)

    deg0, deg1 = _deg_kernel(ei)
    g1, dinv = _mm1(x, W1, deg0, deg1)
    s1a, s1b = _scatter_hid(g1, ei)
    g2 = _mid(s1a, s1b, g1, dinv, b1.reshape(1, HID), W2)
    s2a, s2b = _scatter_lat(g2, ei)
    return _fin(s2a, s2b, g2, dinv, b2.reshape(1, LAT))


# speedup vs baseline: 1.1884x; 1.0594x over previous
import functools

import jax
import jax.numpy as jnp
from jax import lax
from jax.experimental import pallas as pl
from jax.experimental.pallas import tpu as pltpu
from jax.experimental.pallas import tpu_sc as plsc

N = 10000
E = 160000
D_IN = 256
HID = 128
LAT = 64

NC = 2
NS = 16
L = 16
NW = NC * NS

N_PAD = 10240
E_PAD = 163840
CHUNK = 128
EPW = E_PAD // NW
NCHUNK = EPW // CHUNK
ROWS_PW = N_PAD // NS
FLUSH = 128

_MESH = plsc.VectorSubcoreMesh(
    core_axis_name="c", subcore_axis_name="s", num_cores=NC, num_subcores=NS)


EROWS = E // CHUNK
RROWS = EROWS - (NW - 1) * NCHUNK


def _stage_edges(ei_hbm, wid, idx_v, which, want_src):
    @pl.when(wid < NW - 1)
    def _():
        pltpu.sync_copy(ei_hbm.at[which, pl.ds(wid * NCHUNK, NCHUNK)], idx_v)

    @pl.when(wid == NW - 1)
    def _():
        pltpu.sync_copy(
            ei_hbm.at[which, pl.ds((NW - 1) * NCHUNK, RROWS)],
            idx_v.at[pl.ds(0, RROWS)])
        iota = lax.iota(jnp.int32, L)
        kl = CHUNK // L

        def pbody(i, carry):
            vals = i * L + iota
            if not want_src:
                vals = N + (vals & 127)
            idx_v[i // kl, pl.ds((i % kl) * L, L)] = vals
            return carry
        lax.fori_loop(RROWS * kl, NCHUNK * kl, pbody, 0)


def _stage_flat(eif_hbm, wid, v, which, want_src):
    iota = lax.iota(jnp.int32, L)
    epw_real = E - (NW - 1) * EPW

    @pl.when(wid < NW - 1)
    def _():
        pltpu.sync_copy(eif_hbm.at[which, pl.ds(wid * EPW, EPW)], v)

    @pl.when(wid == NW - 1)
    def _():
        pltpu.sync_copy(eif_hbm.at[which, pl.ds((NW - 1) * EPW, epw_real)],
                        v.at[pl.ds(0, epw_real)])

        def pbody(i, carry):
            vals = i * L + iota
            if not want_src:
                vals = N + (vals & 127)
            v[pl.ds(i * L, L)] = vals
            return carry
        lax.fori_loop(epw_real // L, EPW // L, pbody, 0)


def _deg_body(ei_hbm, out0, out1, dst_v, hist_v, acc_v, tmp_v, shared):
    cid = lax.axis_index("c")
    sid = lax.axis_index("s")
    wid = cid * NS + sid
    _stage_flat(ei_hbm, wid, dst_v, 1, False)

    zero = jnp.zeros((L,), jnp.float32)
    ones = jnp.ones((L,), jnp.float32)

    def zbody(i, carry):
        hist_v[pl.ds(i * L, L)] = zero
        return carry
    lax.fori_loop(0, N_PAD // L, zbody, 0)

    def hbody(i, carry):
        idx = dst_v[pl.ds(i * L, L)]
        plsc.addupdate_scatter(hist_v, [idx], ones)
        return carry
    lax.fori_loop(0, EPW // L, hbody, 0)

    pltpu.sync_copy(hist_v, shared.at[sid])
    plsc.subcore_barrier()

    base = sid * ROWS_PW
    pltpu.sync_copy(shared.at[:, pl.ds(base, ROWS_PW)], tmp_v)

    def abody(i, carry):
        s = tmp_v[0, pl.ds(i * L, L)]
        for t in range(1, NS):
            s = s + tmp_v[t, pl.ds(i * L, L)]
        acc_v[pl.ds(i * L, L)] = s
        return carry
    lax.fori_loop(0, ROWS_PW // L, abody, 0)

    @pl.when(cid == 0)
    def _():
        pltpu.sync_copy(acc_v, out0.at[pl.ds(base, ROWS_PW)])

    @pl.when(cid == 1)
    def _():
        pltpu.sync_copy(acc_v, out1.at[pl.ds(base, ROWS_PW)])


_deg_kernel = functools.partial(
    pl.kernel, mesh=_MESH,
    compiler_params=pltpu.CompilerParams(needs_layout_passes=False),
    out_type=(jax.ShapeDtypeStruct((N_PAD,), jnp.float32),
              jax.ShapeDtypeStruct((N_PAD,), jnp.float32)),
    scratch_types=[
        pltpu.VMEM((EPW,), jnp.int32),
        pltpu.VMEM((N_PAD,), jnp.float32),
        pltpu.VMEM((ROWS_PW,), jnp.float32),
        pltpu.VMEM((NS, ROWS_PW), jnp.float32),
        pltpu.VMEM_SHARED((NS, N_PAD), jnp.float32),
    ])(_deg_body)


def _make_scatter(D, nbuf, chunk=CHUNK):
    DL = D // L
    nchunk = EPW // chunk

    def body(g_hbm, eif_hbm, out0, out1, src_v, dst_v, *rest):
        gbufs = rest[:nbuf]
        sems = rest[nbuf:2 * nbuf]
        acc = rest[2 * nbuf]
        cid = lax.axis_index("c")
        sid = lax.axis_index("s")
        wid = cid * NS + sid
        _stage_flat(eif_hbm, wid, src_v, 0, True)
        _stage_flat(eif_hbm, wid, dst_v, 1, False)

        zero = jnp.zeros((L,), jnp.float32)
        gb0 = gbufs[0]

        def zbody(i, carry):
            gb0[i // DL, pl.ds((i % DL) * L, L)] = zero
            return carry
        lax.fori_loop(0, chunk * DL, zbody, 0)

        base = sid * ROWS_PW
        for h in range(ROWS_PW // chunk):
            pltpu.sync_copy(gb0, acc.at[pl.ds(base + h * chunk, chunk)])
        plsc.subcore_barrier()

        for b in range(nbuf):
            pltpu.async_copy(g_hbm.at[src_v.at[pl.ds(chunk * b, chunk)]],
                             gbufs[b], sems[b])

        def step(j, carry):
            for b in range(nbuf):
                t = nbuf * j + b
                pltpu.make_async_copy(
                    g_hbm.at[src_v.at[pl.ds(chunk * t, chunk)]],
                    gbufs[b], sems[b]).wait()
                pltpu.sync_copy(gbufs[b],
                                acc.at[dst_v.at[pl.ds(chunk * t, chunk)]],
                                add=True)

                @pl.when(t + nbuf < nchunk)
                def _():
                    pltpu.async_copy(
                        g_hbm.at[src_v.at[pl.ds(chunk * (t + nbuf), chunk)]],
                        gbufs[b], sems[b])
            return carry
        lax.fori_loop(0, nchunk // nbuf, step, 0)

        plsc.subcore_barrier()
        for h in range(ROWS_PW // chunk):
            pltpu.sync_copy(acc.at[pl.ds(base + h * chunk, chunk)], gb0)

            @pl.when(cid == 0)
            def _():
                pltpu.sync_copy(gb0, out0.at[pl.ds(base + h * chunk, chunk)])

            @pl.when(cid == 1)
            def _():
                pltpu.sync_copy(gb0, out1.at[pl.ds(base + h * chunk, chunk)])

    return functools.partial(
        pl.kernel, mesh=_MESH,
        compiler_params=pltpu.CompilerParams(use_tc_tiling_on_sc=False),
        out_type=(jax.ShapeDtypeStruct((N_PAD, D), jnp.float32),
                  jax.ShapeDtypeStruct((N_PAD, D), jnp.float32)),
        scratch_types=[
            pltpu.VMEM((EPW,), jnp.int32),
            pltpu.VMEM((EPW,), jnp.int32),
        ] + [pltpu.VMEM((chunk, D), jnp.float32)] * nbuf
          + [pltpu.SemaphoreType.DMA] * nbuf
          + [pltpu.VMEM_SHARED((N_PAD, D), jnp.float32)],
    )(body)


_scatter_hid = _make_scatter(HID, nbuf=4, chunk=64)
_scatter_lat = _make_scatter(LAT, nbuf=8)


RB = 1024
_GRID = (N_PAD // RB,)


def _rows(d):
    return pl.BlockSpec((RB, d), lambda i: (i, 0))


def _vec():
    return pl.BlockSpec((RB,), lambda i: (i,))


def _full(*shape):
    return pl.BlockSpec(shape, lambda i: tuple(0 for _ in shape))


def _mm1_body(x_ref, w_ref, d0_ref, d1_ref, g_ref, dinv_ref):
    dinv = lax.rsqrt(d0_ref[:] + d1_ref[:] + 1.0)
    h = jnp.dot(x_ref[:], w_ref[:], preferred_element_type=jnp.float32)
    g_ref[:] = h * dinv[:, None]
    dinv_ref[:] = dinv


_mm1 = pl.pallas_call(
    _mm1_body,
    grid=_GRID,
    in_specs=[_rows(D_IN), _full(D_IN, HID), _vec(), _vec()],
    out_specs=(_rows(HID), _vec()),
    out_shape=(jax.ShapeDtypeStruct((N_PAD, HID), jnp.float32),
               jax.ShapeDtypeStruct((N_PAD,), jnp.float32)),
)


def _mid_body(s0_ref, s1_ref, g1_ref, dinv_ref, b1_ref, w2_ref, g2_ref):
    dinv = dinv_ref[:]
    out1 = (s0_ref[:] + s1_ref[:] + g1_ref[:]) * dinv[:, None] + b1_ref[:]
    x2 = jnp.maximum(out1, 0.0)
    h2 = jnp.dot(x2, w2_ref[:], preferred_element_type=jnp.float32)
    g2_ref[:] = h2 * dinv[:, None]


_mid = pl.pallas_call(
    _mid_body,
    grid=_GRID,
    in_specs=[_rows(HID), _rows(HID), _rows(HID), _vec(),
              _full(1, HID), _full(HID, LAT)],
    out_specs=_rows(LAT),
    out_shape=jax.ShapeDtypeStruct((N_PAD, LAT), jnp.float32),
)


def _fin_body(s0_ref, s1_ref, g2_ref, dinv_ref, b2_ref, mean_ref, logvar_ref):
    dinv = dinv_ref[:]
    out = (s0_ref[:] + s1_ref[:] + g2_ref[:]) * dinv[:, None] + b2_ref[:]
    mean_ref[:] = out[:, : LAT // 2]
    logvar_ref[:] = out[:, LAT // 2:]


_fin = pl.pallas_call(
    _fin_body,
    grid=_GRID,
    in_specs=[_rows(LAT), _rows(LAT), _rows(LAT), _vec(), _full(1, LAT)],
    out_specs=(pl.BlockSpec((RB, LAT // 2), lambda i: (i, 0)),
               pl.BlockSpec((RB, LAT // 2), lambda i: (i, 0))),
    out_shape=(jax.ShapeDtypeStruct((N, LAT // 2), jnp.float32),
               jax.ShapeDtypeStruct((N, LAT // 2), jnp.float32)),
)


def kernel(x, edge_index, W1, b1, W2, b2):
    ei = edge_index.astype(jnp.int32)

    deg0, deg1 = _deg_kernel(ei)
    g1, dinv = _mm1(x, W1, deg0, deg1)
    s1a, s1b = _scatter_hid(g1, ei)
    g2 = _mid(s1a, s1b, g1, dinv, b1.reshape(1, HID), W2)
    s2a, s2b = _scatter_lat(g2, ei)
    return _fin(s2a, s2b, g2, dinv, b2.reshape(1, LAT))
